# Initial kernel scaffold; baseline (speedup 1.0000x reference)
import jax
import jax.numpy as jnp
from jax.experimental import pallas as pl


def kernel(z, codebooks):
    raise NotImplementedError("write your pallas kernel here")


# speedup vs baseline: 2.4003x; 2.4003x over previous
import jax
import jax.numpy as jnp
from jax.experimental import pallas as pl
from jax.experimental.pallas import tpu as pltpu

_NQ = 8
_K = 1024
_D = 256
_BLK = 2048


def _rvq_block_kernel(z_ref, cb_ref, q_ref, idx_ref, loss_ref):
    z = z_ref[...]
    residual = z
    quantized = jnp.zeros_like(z)
    loss_sum = jnp.float32(0.0)
    lane_iota = jax.lax.broadcasted_iota(jnp.int32, (_BLK, _K), 1)
    for i in range(_NQ):
        cb = cb_ref[i]
        rnorm = jnp.sum(residual * residual, axis=1, keepdims=True)
        cbnorm = jnp.sum(cb * cb, axis=1)
        prod = jax.lax.dot_general(
            residual, cb, (((1,), (1,)), ((), ())),
            preferred_element_type=jnp.float32)
        d2 = (rnorm - 2.0 * prod) + cbnorm[None, :]
        min_d = jnp.min(d2, axis=1, keepdims=True)
        idx = jnp.min(jnp.where(d2 == min_d, lane_iota, _K), axis=1)
        onehot = (lane_iota == idx[:, None]).astype(jnp.float32)
        ql = jnp.dot(onehot, cb, preferred_element_type=jnp.float32)
        quantized = quantized + ql
        residual = residual - ql
        loss_sum = loss_sum + jnp.sum(residual * residual)
        idx_ref[i, :] = idx
    q_ref[...] = quantized

    @pl.when(pl.program_id(0) == 0)
    def _init():
        loss_ref[0, 0] = jnp.float32(0.0)

    loss_ref[0, 0] += loss_sum


def kernel(z, codebooks):
    B, T, D = z.shape
    ntok = B * T
    zf = z.reshape(ntok, D)
    nblocks = ntok // _BLK
    q, idx, loss = pl.pallas_call(
        _rvq_block_kernel,
        grid=(nblocks,),
        in_specs=[
            pl.BlockSpec((_BLK, _D), lambda i: (i, 0)),
            pl.BlockSpec((_NQ, _K, _D), lambda i: (0, 0, 0)),
        ],
        out_specs=[
            pl.BlockSpec((_BLK, _D), lambda i: (i, 0)),
            pl.BlockSpec((_NQ, _BLK), lambda i: (0, i)),
            pl.BlockSpec(memory_space=pltpu.SMEM),
        ],
        out_shape=[
            jax.ShapeDtypeStruct((ntok, _D), jnp.float32),
            jax.ShapeDtypeStruct((_NQ, ntok), jnp.int32),
            jax.ShapeDtypeStruct((1, 1), jnp.float32),
        ],
    )(zf, codebooks)
    quantized_st = q.reshape(B, T, D)
    indices = idx.reshape(_NQ, B, T).transpose(1, 0, 2)
    commitment_loss = loss[0, 0] / jnp.float32(_NQ * ntok * _D)
    return quantized_st, indices, commitment_loss
